# Initial kernel scaffold; baseline (speedup 1.0000x reference)
import jax
import jax.numpy as jnp
from jax.experimental import pallas as pl


def kernel(x, edge_index, W1, att_src1, att_dst1, b1, W2, att_src2, att_dst2, b2, Wh, bh):
    raise NotImplementedError("write your pallas kernel here")


# speedup vs baseline: 21.2176x; 21.2176x over previous
import functools
import jax
import jax.numpy as jnp
from jax import lax
from jax.experimental import pallas as pl
from jax.experimental.pallas import tpu as pltpu
from jax.experimental.pallas import tpu_sc as plsc

N = 10000
NP = 10112
E = 320000
D = 128
NC, NS, NW = 2, 16, 32
B = 128
CH = 81
EP = NW * B * CH
RPT = NP // NS

_S = jax.ShapeDtypeStruct
_NLP = pltpu.CompilerParams(needs_layout_passes=False)


@functools.lru_cache(maxsize=None)
def _mesh():
    return plsc.VectorSubcoreMesh(core_axis_name="c", subcore_axis_name="s")


def _lazy_kernel(out_type, scratch_types):
    def deco(fn):
        @functools.lru_cache(maxsize=None)
        def build():
            return pl.kernel(fn, out_type=out_type, mesh=_mesh(),
                             scratch_types=list(scratch_types),
                             compiler_params=_NLP)

        def call(*args):
            return build()(*args)
        return call
    return deco


def _tile_slices(s):
    out = []
    done = 0
    while done < RPT:
        n = min(B, RPT - done)
        out.append((done, n))
        done += n
    return out


def _tc1_body(x_ref, w_ref, a_ref, h_ref, aa_ref):
    h = jnp.dot(x_ref[...], w_ref[...], preferred_element_type=jnp.float32)
    h_ref[...] = h
    aa_ref[...] = jnp.dot(h, a_ref[...], preferred_element_type=jnp.float32)


def _tc1(xp, W1, A1):
    BN = 1264
    return pl.pallas_call(
        _tc1_body,
        grid=(NP // BN,),
        in_specs=[pl.BlockSpec((BN, D), lambda i: (i, 0)),
                  pl.BlockSpec((D, D), lambda i: (0, 0)),
                  pl.BlockSpec((D, 16), lambda i: (0, 0))],
        out_specs=[pl.BlockSpec((BN, D), lambda i: (i, 0)),
                   pl.BlockSpec((BN, 16), lambda i: (i, 0))],
        out_shape=[_S((NP, D), jnp.float32), _S((NP, 16), jnp.float32)],
    )(xp, W1, A1)


def _tcadd_body(a_ref, b_ref, o_ref):
    o_ref[...] = a_ref[...] + b_ref[...]


def _tcadd(a, b):
    BN = 1264
    n, m = a.shape
    return pl.pallas_call(
        _tcadd_body,
        grid=(n // BN,),
        in_specs=[pl.BlockSpec((BN, m), lambda i: (i, 0)),
                  pl.BlockSpec((BN, m), lambda i: (i, 0))],
        out_specs=pl.BlockSpec((BN, m), lambda i: (i, 0)),
        out_shape=_S((n, m), jnp.float32),
    )(a, b)


def _tc2_body(oa_ref, ob_ref, bias, w_ref, a_ref, h2_ref, aa2_ref):
    v = oa_ref[...] + ob_ref[...] + bias[...]
    x1 = jnp.where(v > 0, v, jnp.exp(v) - 1.0)
    h2 = jnp.dot(x1, w_ref[...], preferred_element_type=jnp.float32)
    h2_ref[...] = jnp.concatenate(
        [h2, jnp.zeros((h2.shape[0], 96), jnp.float32)], axis=1)
    aa2_ref[...] = jnp.dot(h2, a_ref[...], preferred_element_type=jnp.float32)


def _tc2(oa, ob, b1r, W2, A2p):
    BN = 1264
    return pl.pallas_call(
        _tc2_body,
        grid=(NP // BN,),
        in_specs=[pl.BlockSpec((BN, D), lambda i: (i, 0)),
                  pl.BlockSpec((BN, D), lambda i: (i, 0)),
                  pl.BlockSpec((1, D), lambda i: (0, 0)),
                  pl.BlockSpec((D, 32), lambda i: (0, 0)),
                  pl.BlockSpec((32, 16), lambda i: (0, 0))],
        out_specs=[pl.BlockSpec((BN, D), lambda i: (i, 0)),
                   pl.BlockSpec((BN, 16), lambda i: (i, 0))],
        out_shape=[_S((NP, D), jnp.float32), _S((NP, 16), jnp.float32)],
    )(oa, ob, b1r, W2, A2p)


def _tc3_body(pa_ref, pb_ref, b_ref, w_ref, bh_ref, y_ref):
    v = pa_ref[...] + pb_ref[...] + b_ref[...]
    x2 = jnp.where(v > 0, v, jnp.exp(v) - 1.0)
    y_ref[...] = jnp.dot(x2, w_ref[...],
                         preferred_element_type=jnp.float32) + bh_ref[...]


def _tc3(pa, pb, b2r, Whp, bhp):
    BN = 1264
    return pl.pallas_call(
        _tc3_body,
        grid=(NP // BN,),
        in_specs=[pl.BlockSpec((BN, 32), lambda i: (i, 0)),
                  pl.BlockSpec((BN, 32), lambda i: (i, 0)),
                  pl.BlockSpec((1, 32), lambda i: (0, 0)),
                  pl.BlockSpec((32, 8), lambda i: (0, 0)),
                  pl.BlockSpec((1, 8), lambda i: (0, 0))],
        out_specs=pl.BlockSpec((BN, 8), lambda i: (i, 0)),
        out_shape=_S((NP, 8), jnp.float32),
    )(pa, pb, b2r, Whp, bhp)


@_lazy_kernel(
    out_type=[_S((EP, 8), jnp.float32),
              _S((NP, 8), jnp.float32),
              _S((NP, 8), jnp.float32)],
    scratch_types=[pltpu.VMEM((B,), jnp.int32),
                   pltpu.VMEM((B,), jnp.int32),
                   pltpu.VMEM((B, 8), jnp.float32),
                   pltpu.VMEM((B, 8), jnp.float32),
                   pltpu.VMEM_SHARED((NP, 8), jnp.float32),
                   pltpu.VMEM_SHARED((NP, 8), jnp.float32),
                   pltpu.VMEM_SHARED((NP, 8), jnp.float32)],
)
def _sca(src_ref, dst_ref, as_ref, ad_ref,
         s_ref, dena_ref, denb_ref,
         idx_s, idx_d, asr, adr, as_sh, ad_sh, den_sh):
    c = lax.axis_index("c")
    s = lax.axis_index("s")
    w = s * NC + c
    ln = lax.iota(jnp.int32, 16)
    row_off = ln >> 3
    col = ln & 7
    zero16 = jnp.zeros((16,), jnp.float32)

    for (o, n) in _tile_slices(s):
        r = pl.ds(s * RPT + o, n)
        pltpu.sync_copy(as_ref.at[r], asr.at[pl.ds(0, n)])
        pltpu.sync_copy(asr.at[pl.ds(0, n)], as_sh.at[r])
        pltpu.sync_copy(ad_ref.at[r], adr.at[pl.ds(0, n)])
        pltpu.sync_copy(adr.at[pl.ds(0, n)], ad_sh.at[r])

    def zrow(j, _):
        plsc.store_scatter(asr, [row_off + 2 * j, col], zero16)
        return 0
    lax.fori_loop(0, 64, zrow, 0, unroll=8)
    for (o, n) in _tile_slices(s):
        pltpu.sync_copy(asr.at[pl.ds(0, n)],
                        den_sh.at[pl.ds(s * RPT + o, n)])
    plsc.subcore_barrier()

    def chunk(k, _):
        base = (w * CH + k) * B
        pltpu.sync_copy(src_ref.at[pl.ds(base, B)], idx_s)
        pltpu.sync_copy(dst_ref.at[pl.ds(base, B)], idx_d)
        pltpu.sync_copy(as_sh.at[idx_s], asr)
        pltpu.sync_copy(ad_sh.at[idx_d], adr)

        def grp(j, _):
            r = row_off + 2 * j
            a = (plsc.load_gather(asr, [r, col])
                 + plsc.load_gather(adr, [r, col]))
            a = jnp.exp(jnp.where(a >= 0, a, 0.2 * a))
            plsc.store_scatter(asr, [r, col], a)
            return 0
        lax.fori_loop(0, 64, grp, 0, unroll=8)
        pltpu.sync_copy(asr, s_ref.at[pl.ds(base, B)])
        pltpu.sync_copy(asr, den_sh.at[idx_d], add=True)
        return 0
    lax.fori_loop(0, CH, chunk, 0)
    plsc.subcore_barrier()

    for (o, n) in _tile_slices(s):
        r = pl.ds(s * RPT + o, n)
        pltpu.sync_copy(den_sh.at[r], asr.at[pl.ds(0, n)])

        @pl.when(c == 0)
        def _():
            pltpu.sync_copy(asr.at[pl.ds(0, n)], dena_ref.at[r])

        @pl.when(c == 1)
        def _():
            pltpu.sync_copy(asr.at[pl.ds(0, n)], denb_ref.at[r])


@_lazy_kernel(
    out_type=_S((EP, 8), jnp.float32),
    scratch_types=[pltpu.VMEM((B,), jnp.int32),
                   pltpu.VMEM((B, 8), jnp.float32),
                   pltpu.VMEM((B, 8), jnp.float32),
                   pltpu.VMEM_SHARED((NP, 8), jnp.float32)],
)
def _sccoef(dst_ref, s_ref, den_ref, coef_ref,
            idx_d, srow, drow, den_sh):
    c = lax.axis_index("c")
    s = lax.axis_index("s")
    w = s * NC + c
    ln = lax.iota(jnp.int32, 16)
    row_off = ln >> 3
    col = ln & 7

    for (o, n) in _tile_slices(s):
        r = pl.ds(s * RPT + o, n)
        pltpu.sync_copy(den_ref.at[r], drow.at[pl.ds(0, n)])
        pltpu.sync_copy(drow.at[pl.ds(0, n)], den_sh.at[r])
    plsc.subcore_barrier()

    def chunk(k, _):
        base = (w * CH + k) * B
        pltpu.sync_copy(dst_ref.at[pl.ds(base, B)], idx_d)
        pltpu.sync_copy(den_sh.at[idx_d], drow)
        pltpu.sync_copy(s_ref.at[pl.ds(base, B)], srow)

        def grp(j, _):
            r = row_off + 2 * j
            v = plsc.load_gather(srow, [r, col]) / plsc.load_gather(
                drow, [r, col])
            plsc.store_scatter(srow, [r, col], v)
            return 0
        lax.fori_loop(0, 64, grp, 0, unroll=8)
        pltpu.sync_copy(srow, coef_ref.at[pl.ds(base, B)])
        return 0
    lax.fori_loop(0, CH, chunk, 0)


def _make_scb(nh, cw):
    @_lazy_kernel(
        out_type=[_S((NP, D), jnp.float32), _S((NP, D), jnp.float32)],
        scratch_types=[pltpu.VMEM((B,), jnp.int32),
                       pltpu.VMEM((B,), jnp.int32),
                       pltpu.VMEM((B, D), jnp.float32),
                       pltpu.VMEM((B, 8), jnp.float32),
                       pltpu.VMEM_SHARED((NP, D), jnp.float32)],
    )
    def _scb(src_ref, dst_ref, coef_ref, h_ref,
             oa_ref, ob_ref,
             idx_s, idx_d, hrows, crow, out_sh):
        c = lax.axis_index("c")
        s = lax.axis_index("s")
        w = s * NC + c
        zero16 = jnp.zeros((16,), jnp.float32)

        def zrow(i, _):
            for j in range(8):
                hrows[i, pl.ds(16 * j, 16)] = zero16
            return 0
        lax.fori_loop(0, B, zrow, 0, unroll=4)
        for (o, n) in _tile_slices(s):
            pltpu.sync_copy(hrows.at[pl.ds(0, n)],
                            out_sh.at[pl.ds(s * RPT + o, n)])
        plsc.subcore_barrier()

        def chunk(k, _):
            base = (w * CH + k) * B
            pltpu.sync_copy(src_ref.at[pl.ds(base, B)], idx_s)
            pltpu.sync_copy(dst_ref.at[pl.ds(base, B)], idx_d)
            pltpu.sync_copy(h_ref.at[idx_s], hrows)
            pltpu.sync_copy(coef_ref.at[pl.ds(base, B)], crow)

            def wrow(b, _):
                bf = jnp.full((16,), b, jnp.int32)
                for h in range(nh):
                    c16 = plsc.load_gather(
                        crow, [bf, jnp.full((16,), h, jnp.int32)])
                    for q in range(cw // 16):
                        o = h * cw + 16 * q
                        hrows[b, pl.ds(o, 16)] = hrows[b, pl.ds(o, 16)] * c16
                return 0
            lax.fori_loop(0, B, wrow, 0, unroll=2)
            pltpu.sync_copy(hrows, out_sh.at[idx_d], add=True)
            return 0
        lax.fori_loop(0, CH, chunk, 0)
        plsc.subcore_barrier()

        for (o, n) in _tile_slices(s):
            r = pl.ds(s * RPT + o, n)
            pltpu.sync_copy(out_sh.at[r], hrows.at[pl.ds(0, n)])

            @pl.when(c == 0)
            def _():
                pltpu.sync_copy(hrows.at[pl.ds(0, n)], oa_ref.at[r])

            @pl.when(c == 1)
            def _():
                pltpu.sync_copy(hrows.at[pl.ds(0, n)], ob_ref.at[r])
    return _scb


_scb1 = _make_scb(8, 16)
_scb2 = _make_scb(1, 32)


def _block_diag(att, heads, ch):
    a = att.reshape(heads, ch)
    return (a[:, :, None] * jnp.eye(heads, dtype=a.dtype)[:, None, :]
            ).reshape(heads * ch, heads)


def kernel(x, edge_index, W1, att_src1, att_dst1, b1,
           W2, att_src2, att_dst2, b2, Wh, bh):
    loop = jnp.arange(N, dtype=edge_index.dtype)
    src = jnp.concatenate([edge_index[0], loop,
                           jnp.zeros((EP - E - N,), jnp.int32)])
    dst = jnp.concatenate([edge_index[1], loop,
                           jnp.full((EP - E - N,), N, jnp.int32)])
    xp = jnp.pad(x, ((0, NP - N), (0, 0)))

    A1 = jnp.concatenate([_block_diag(att_src1, 8, 16),
                          _block_diag(att_dst1, 8, 16)], axis=1)
    h1, aa1 = _tc1(xp, W1, A1)
    as1 = aa1[:, :8]
    ad1 = aa1[:, 8:]

    s1, d1a, d1b = _sca(src, dst, as1, ad1)
    den1 = _tcadd(d1a, d1b)
    coef1 = _sccoef(dst, s1, den1)
    oa, ob = _scb1(src, dst, coef1, h1)

    A2p = jnp.zeros((32, 16), jnp.float32)
    A2p = A2p.at[:, 0].set(att_src2.reshape(32)).at[:, 8].set(
        att_dst2.reshape(32))
    h2p, aa2 = _tc2(oa, ob, b1.reshape(1, D), W2, A2p)
    as2 = aa2[:, :8]
    ad2 = aa2[:, 8:]

    s2, d2a, d2b = _sca(src, dst, as2, ad2)
    den2 = _tcadd(d2a, d2b)
    coef2 = _sccoef(dst, s2, den2)
    o2a, o2b = _scb2(src, dst, coef2, h2p)

    Whp = jnp.pad(Wh, ((0, 0), (0, 7)))
    bhp = jnp.broadcast_to(bh.reshape(1, 1), (1, 8))
    y8 = _tc3(o2a[:, :32], o2b[:, :32], b2.reshape(1, 32), Whp, bhp)
    return y8[:N, 0:1]


# speedup vs baseline: 27.2791x; 1.2857x over previous
import functools
import jax
import jax.numpy as jnp
from jax import lax
from jax.experimental import pallas as pl
from jax.experimental.pallas import tpu as pltpu
from jax.experimental.pallas import tpu_sc as plsc

N = 10000
NP = 10112
E = 320000
D = 128
NC, NS, NW = 2, 16, 32
B = 128
CH = 81
EP = NW * B * CH
RPT = NP // NS

_S = jax.ShapeDtypeStruct
_NLP = pltpu.CompilerParams(needs_layout_passes=False)


@functools.lru_cache(maxsize=None)
def _mesh():
    return plsc.VectorSubcoreMesh(core_axis_name="c", subcore_axis_name="s")


def _lazy_kernel(out_type, scratch_types):
    def deco(fn):
        @functools.lru_cache(maxsize=None)
        def build():
            return pl.kernel(fn, out_type=out_type, mesh=_mesh(),
                             scratch_types=list(scratch_types),
                             compiler_params=_NLP)

        def call(*args):
            return build()(*args)
        return call
    return deco


def _tile_slices(s):
    out = []
    done = 0
    while done < RPT:
        n = min(B, RPT - done)
        out.append((done, n))
        done += n
    return out


def _tc1_body(x_ref, w_ref, a_ref, h_ref, aa_ref):
    h = jnp.dot(x_ref[...], w_ref[...], preferred_element_type=jnp.float32)
    h_ref[...] = h
    aa_ref[...] = jnp.dot(h, a_ref[...], preferred_element_type=jnp.float32)


def _tc1(xp, W1, A1):
    BN = 1264
    return pl.pallas_call(
        _tc1_body,
        grid=(NP // BN,),
        in_specs=[pl.BlockSpec((BN, D), lambda i: (i, 0)),
                  pl.BlockSpec((D, D), lambda i: (0, 0)),
                  pl.BlockSpec((D, 16), lambda i: (0, 0))],
        out_specs=[pl.BlockSpec((BN, D), lambda i: (i, 0)),
                   pl.BlockSpec((BN, 16), lambda i: (i, 0))],
        out_shape=[_S((NP, D), jnp.float32), _S((NP, 16), jnp.float32)],
    )(xp, W1, A1)


def _tcadd_body(a_ref, b_ref, o_ref):
    o_ref[...] = a_ref[...] + b_ref[...]


def _tcadd(a, b):
    BN = 1264
    n, m = a.shape
    return pl.pallas_call(
        _tcadd_body,
        grid=(n // BN,),
        in_specs=[pl.BlockSpec((BN, m), lambda i: (i, 0)),
                  pl.BlockSpec((BN, m), lambda i: (i, 0))],
        out_specs=pl.BlockSpec((BN, m), lambda i: (i, 0)),
        out_shape=_S((n, m), jnp.float32),
    )(a, b)


def _tc2_body(oa_ref, ob_ref, da_ref, db_ref, bias, w_ref, a_ref,
              h2_ref, aa2_ref):
    den = da_ref[...] + db_ref[...]
    bn = den.shape[0]
    denx = jnp.broadcast_to(den.reshape(bn, 8, 1),
                            (bn, 8, 16)).reshape(bn, D)
    v = (oa_ref[...] + ob_ref[...]) / denx + bias[...]
    x1 = jnp.where(v > 0, v, jnp.exp(v) - 1.0)
    h2 = jnp.dot(x1, w_ref[...], preferred_element_type=jnp.float32)
    h2_ref[...] = jnp.concatenate(
        [h2, jnp.zeros((h2.shape[0], 96), jnp.float32)], axis=1)
    aa2_ref[...] = jnp.dot(h2, a_ref[...], preferred_element_type=jnp.float32)


def _tc2(oa, ob, da, db, b1r, W2, A2p):
    BN = 1264
    return pl.pallas_call(
        _tc2_body,
        grid=(NP // BN,),
        in_specs=[pl.BlockSpec((BN, D), lambda i: (i, 0)),
                  pl.BlockSpec((BN, D), lambda i: (i, 0)),
                  pl.BlockSpec((BN, 8), lambda i: (i, 0)),
                  pl.BlockSpec((BN, 8), lambda i: (i, 0)),
                  pl.BlockSpec((1, D), lambda i: (0, 0)),
                  pl.BlockSpec((D, 32), lambda i: (0, 0)),
                  pl.BlockSpec((32, 16), lambda i: (0, 0))],
        out_specs=[pl.BlockSpec((BN, D), lambda i: (i, 0)),
                   pl.BlockSpec((BN, 16), lambda i: (i, 0))],
        out_shape=[_S((NP, D), jnp.float32), _S((NP, 16), jnp.float32)],
    )(oa, ob, da, db, b1r, W2, A2p)


def _tc3_body(pa_ref, pb_ref, da_ref, db_ref, b_ref, w_ref, bh_ref, y_ref):
    den = da_ref[...] + db_ref[...]
    v = (pa_ref[...] + pb_ref[...]) / den[:, 0:1] + b_ref[...]
    x2 = jnp.where(v > 0, v, jnp.exp(v) - 1.0)
    y_ref[...] = jnp.dot(x2, w_ref[...],
                         preferred_element_type=jnp.float32) + bh_ref[...]


def _tc3(pa, pb, da, db, b2r, Whp, bhp):
    BN = 1264
    return pl.pallas_call(
        _tc3_body,
        grid=(NP // BN,),
        in_specs=[pl.BlockSpec((BN, 32), lambda i: (i, 0)),
                  pl.BlockSpec((BN, 32), lambda i: (i, 0)),
                  pl.BlockSpec((BN, 8), lambda i: (i, 0)),
                  pl.BlockSpec((BN, 8), lambda i: (i, 0)),
                  pl.BlockSpec((1, 32), lambda i: (0, 0)),
                  pl.BlockSpec((32, 8), lambda i: (0, 0)),
                  pl.BlockSpec((1, 8), lambda i: (0, 0))],
        out_specs=pl.BlockSpec((BN, 8), lambda i: (i, 0)),
        out_shape=_S((NP, 8), jnp.float32),
    )(pa, pb, da, db, b2r, Whp, bhp)


@_lazy_kernel(
    out_type=[_S((EP, 8), jnp.float32),
              _S((NP, 8), jnp.float32),
              _S((NP, 8), jnp.float32)],
    scratch_types=[pltpu.VMEM((B,), jnp.int32),
                   pltpu.VMEM((B,), jnp.int32),
                   pltpu.VMEM((B, 8), jnp.float32),
                   pltpu.VMEM((B, 8), jnp.float32),
                   pltpu.VMEM_SHARED((NP, 8), jnp.float32),
                   pltpu.VMEM_SHARED((NP, 8), jnp.float32),
                   pltpu.VMEM_SHARED((NP, 8), jnp.float32)],
)
def _sca(src_ref, dst_ref, as_ref, ad_ref,
         s_ref, dena_ref, denb_ref,
         idx_s, idx_d, asr, adr, as_sh, ad_sh, den_sh):
    c = lax.axis_index("c")
    s = lax.axis_index("s")
    w = s * NC + c
    ln = lax.iota(jnp.int32, 16)
    row_off = ln >> 3
    col = ln & 7
    zero16 = jnp.zeros((16,), jnp.float32)

    for (o, n) in _tile_slices(s):
        r = pl.ds(s * RPT + o, n)
        pltpu.sync_copy(as_ref.at[r], asr.at[pl.ds(0, n)])
        pltpu.sync_copy(asr.at[pl.ds(0, n)], as_sh.at[r])
        pltpu.sync_copy(ad_ref.at[r], adr.at[pl.ds(0, n)])
        pltpu.sync_copy(adr.at[pl.ds(0, n)], ad_sh.at[r])

    def zrow(j, _):
        plsc.store_scatter(asr, [row_off + 2 * j, col], zero16)
        return 0
    lax.fori_loop(0, 64, zrow, 0, unroll=8)
    for (o, n) in _tile_slices(s):
        pltpu.sync_copy(asr.at[pl.ds(0, n)],
                        den_sh.at[pl.ds(s * RPT + o, n)])
    plsc.subcore_barrier()

    def chunk(k, _):
        base = (w * CH + k) * B
        pltpu.sync_copy(src_ref.at[pl.ds(base, B)], idx_s)
        pltpu.sync_copy(dst_ref.at[pl.ds(base, B)], idx_d)
        pltpu.sync_copy(as_sh.at[idx_s], asr)
        pltpu.sync_copy(ad_sh.at[idx_d], adr)

        def grp(j, _):
            r = row_off + 2 * j
            a = (plsc.load_gather(asr, [r, col])
                 + plsc.load_gather(adr, [r, col]))
            a = jnp.exp(jnp.where(a >= 0, a, 0.2 * a))
            plsc.store_scatter(asr, [r, col], a)
            return 0
        lax.fori_loop(0, 64, grp, 0, unroll=8)
        pltpu.sync_copy(asr, s_ref.at[pl.ds(base, B)])
        pltpu.sync_copy(asr, den_sh.at[idx_d], add=True)
        return 0
    lax.fori_loop(0, CH, chunk, 0)
    plsc.subcore_barrier()

    for (o, n) in _tile_slices(s):
        r = pl.ds(s * RPT + o, n)
        pltpu.sync_copy(den_sh.at[r], asr.at[pl.ds(0, n)])

        @pl.when(c == 0)
        def _():
            pltpu.sync_copy(asr.at[pl.ds(0, n)], dena_ref.at[r])

        @pl.when(c == 1)
        def _():
            pltpu.sync_copy(asr.at[pl.ds(0, n)], denb_ref.at[r])


@_lazy_kernel(
    out_type=_S((EP, 8), jnp.float32),
    scratch_types=[pltpu.VMEM((B,), jnp.int32),
                   pltpu.VMEM((B, 8), jnp.float32),
                   pltpu.VMEM((B, 8), jnp.float32),
                   pltpu.VMEM_SHARED((NP, 8), jnp.float32)],
)
def _sccoef(dst_ref, s_ref, den_ref, coef_ref,
            idx_d, srow, drow, den_sh):
    c = lax.axis_index("c")
    s = lax.axis_index("s")
    w = s * NC + c
    ln = lax.iota(jnp.int32, 16)
    row_off = ln >> 3
    col = ln & 7

    for (o, n) in _tile_slices(s):
        r = pl.ds(s * RPT + o, n)
        pltpu.sync_copy(den_ref.at[r], drow.at[pl.ds(0, n)])
        pltpu.sync_copy(drow.at[pl.ds(0, n)], den_sh.at[r])
    plsc.subcore_barrier()

    def chunk(k, _):
        base = (w * CH + k) * B
        pltpu.sync_copy(dst_ref.at[pl.ds(base, B)], idx_d)
        pltpu.sync_copy(den_sh.at[idx_d], drow)
        pltpu.sync_copy(s_ref.at[pl.ds(base, B)], srow)

        def grp(j, _):
            r = row_off + 2 * j
            v = plsc.load_gather(srow, [r, col]) / plsc.load_gather(
                drow, [r, col])
            plsc.store_scatter(srow, [r, col], v)
            return 0
        lax.fori_loop(0, 64, grp, 0, unroll=8)
        pltpu.sync_copy(srow, coef_ref.at[pl.ds(base, B)])
        return 0
    lax.fori_loop(0, CH, chunk, 0)


def _make_scb(nh, cw):
    @_lazy_kernel(
        out_type=[_S((NP, D), jnp.float32), _S((NP, D), jnp.float32)],
        scratch_types=[pltpu.VMEM((B,), jnp.int32),
                       pltpu.VMEM((B,), jnp.int32),
                       pltpu.VMEM((B, D), jnp.float32),
                       pltpu.VMEM((B, 8), jnp.float32),
                       pltpu.VMEM_SHARED((NP, D), jnp.float32)],
    )
    def _scb(src_ref, dst_ref, coef_ref, h_ref,
             oa_ref, ob_ref,
             idx_s, idx_d, hrows, crow, out_sh):
        c = lax.axis_index("c")
        s = lax.axis_index("s")
        w = s * NC + c
        zero16 = jnp.zeros((16,), jnp.float32)

        def zrow(i, _):
            for j in range(8):
                hrows[i, pl.ds(16 * j, 16)] = zero16
            return 0
        lax.fori_loop(0, B, zrow, 0, unroll=4)
        for (o, n) in _tile_slices(s):
            pltpu.sync_copy(hrows.at[pl.ds(0, n)],
                            out_sh.at[pl.ds(s * RPT + o, n)])
        plsc.subcore_barrier()

        def chunk(k, _):
            base = (w * CH + k) * B
            pltpu.sync_copy(src_ref.at[pl.ds(base, B)], idx_s)
            pltpu.sync_copy(dst_ref.at[pl.ds(base, B)], idx_d)
            pltpu.sync_copy(h_ref.at[idx_s], hrows)
            pltpu.sync_copy(coef_ref.at[pl.ds(base, B)], crow)

            def wrow(b, _):
                bf = jnp.full((16,), b, jnp.int32)
                for h in range(nh):
                    c16 = plsc.load_gather(
                        crow, [bf, jnp.full((16,), h, jnp.int32)])
                    for q in range(cw // 16):
                        o = h * cw + 16 * q
                        hrows[b, pl.ds(o, 16)] = hrows[b, pl.ds(o, 16)] * c16
                return 0
            lax.fori_loop(0, B, wrow, 0, unroll=2)
            pltpu.sync_copy(hrows, out_sh.at[idx_d], add=True)
            return 0
        lax.fori_loop(0, CH, chunk, 0)
        plsc.subcore_barrier()

        for (o, n) in _tile_slices(s):
            r = pl.ds(s * RPT + o, n)
            pltpu.sync_copy(out_sh.at[r], hrows.at[pl.ds(0, n)])

            @pl.when(c == 0)
            def _():
                pltpu.sync_copy(hrows.at[pl.ds(0, n)], oa_ref.at[r])

            @pl.when(c == 1)
            def _():
                pltpu.sync_copy(hrows.at[pl.ds(0, n)], ob_ref.at[r])
    return _scb


_scb1 = _make_scb(8, 16)
_scb2 = _make_scb(1, 32)


def _block_diag(att, heads, ch):
    a = att.reshape(heads, ch)
    return (a[:, :, None] * jnp.eye(heads, dtype=a.dtype)[:, None, :]
            ).reshape(heads * ch, heads)


def kernel(x, edge_index, W1, att_src1, att_dst1, b1,
           W2, att_src2, att_dst2, b2, Wh, bh):
    loop = jnp.arange(N, dtype=edge_index.dtype)
    src = jnp.concatenate([edge_index[0], loop,
                           jnp.zeros((EP - E - N,), jnp.int32)])
    dst = jnp.concatenate([edge_index[1], loop,
                           jnp.full((EP - E - N,), N, jnp.int32)])
    xp = jnp.pad(x, ((0, NP - N), (0, 0)))

    A1 = jnp.concatenate([_block_diag(att_src1, 8, 16),
                          _block_diag(att_dst1, 8, 16)], axis=1)
    h1, aa1 = _tc1(xp, W1, A1)
    as1 = aa1[:, :8]
    ad1 = aa1[:, 8:]

    s1, d1a, d1b = _sca(src, dst, as1, ad1)
    oa, ob = _scb1(src, dst, s1, h1)

    A2p = jnp.zeros((32, 16), jnp.float32)
    A2p = A2p.at[:, 0].set(att_src2.reshape(32)).at[:, 8].set(
        att_dst2.reshape(32))
    h2p, aa2 = _tc2(oa, ob, d1a, d1b, b1.reshape(1, D), W2, A2p)
    as2 = aa2[:, :8]
    ad2 = aa2[:, 8:]

    s2, d2a, d2b = _sca(src, dst, as2, ad2)
    o2a, o2b = _scb2(src, dst, s2, h2p)

    Whp = jnp.pad(Wh, ((0, 0), (0, 7)))
    bhp = jnp.broadcast_to(bh.reshape(1, 1), (1, 8))
    y8 = _tc3(o2a[:, :32], o2b[:, :32], d2a, d2b, b2.reshape(1, 32),
              Whp, bhp)
    return y8[:N, 0:1]


# speedup vs baseline: 27.3325x; 1.0020x over previous
import functools
import jax
import jax.numpy as jnp
from jax import lax
from jax.experimental import pallas as pl
from jax.experimental.pallas import tpu as pltpu
from jax.experimental.pallas import tpu_sc as plsc

N = 10000
NP = 10112
E = 320000
D = 128
NC, NS, NW = 2, 16, 32
B = 128
CH = 81
EP = NW * B * CH
RPT = NP // NS

_S = jax.ShapeDtypeStruct
_NLP = pltpu.CompilerParams(needs_layout_passes=False)


@functools.lru_cache(maxsize=None)
def _mesh():
    return plsc.VectorSubcoreMesh(core_axis_name="c", subcore_axis_name="s")


def _lazy_kernel(out_type, scratch_types):
    def deco(fn):
        @functools.lru_cache(maxsize=None)
        def build():
            return pl.kernel(fn, out_type=out_type, mesh=_mesh(),
                             scratch_types=list(scratch_types),
                             compiler_params=_NLP)

        def call(*args):
            return build()(*args)
        return call
    return deco


def _tile_slices(s):
    out = []
    done = 0
    while done < RPT:
        n = min(B, RPT - done)
        out.append((done, n))
        done += n
    return out


def _tc1_body(x_ref, w_ref, a_ref, h_ref, aa_ref):
    h = jnp.dot(x_ref[...], w_ref[...], preferred_element_type=jnp.float32)
    h_ref[...] = h
    aa_ref[...] = jnp.dot(h, a_ref[...], preferred_element_type=jnp.float32)


def _tc1(xp, W1, A1):
    BN = 1264
    return pl.pallas_call(
        _tc1_body,
        grid=(NP // BN,),
        in_specs=[pl.BlockSpec((BN, D), lambda i: (i, 0)),
                  pl.BlockSpec((D, D), lambda i: (0, 0)),
                  pl.BlockSpec((D, 16), lambda i: (0, 0))],
        out_specs=[pl.BlockSpec((BN, D), lambda i: (i, 0)),
                   pl.BlockSpec((BN, 16), lambda i: (i, 0))],
        out_shape=[_S((NP, D), jnp.float32), _S((NP, 16), jnp.float32)],
    )(xp, W1, A1)


def _tc2_body(oa_ref, ob_ref, da_ref, db_ref, bias, w_ref, a_ref,
              h2_ref, aa2_ref):
    den = da_ref[...] + db_ref[...]
    bn = den.shape[0]
    denx = jnp.broadcast_to(den.reshape(bn, 8, 1),
                            (bn, 8, 16)).reshape(bn, D)
    v = (oa_ref[...] + ob_ref[...]) / denx + bias[...]
    x1 = jnp.where(v > 0, v, jnp.exp(v) - 1.0)
    h2 = jnp.dot(x1, w_ref[...], preferred_element_type=jnp.float32)
    h2_ref[...] = jnp.concatenate(
        [h2, jnp.zeros((h2.shape[0], 96), jnp.float32)], axis=1)
    aa2_ref[...] = jnp.dot(h2, a_ref[...], preferred_element_type=jnp.float32)


def _tc2(oa, ob, da, db, b1r, W2, A2p):
    BN = 1264
    return pl.pallas_call(
        _tc2_body,
        grid=(NP // BN,),
        in_specs=[pl.BlockSpec((BN, D), lambda i: (i, 0)),
                  pl.BlockSpec((BN, D), lambda i: (i, 0)),
                  pl.BlockSpec((BN, 8), lambda i: (i, 0)),
                  pl.BlockSpec((BN, 8), lambda i: (i, 0)),
                  pl.BlockSpec((1, D), lambda i: (0, 0)),
                  pl.BlockSpec((D, 32), lambda i: (0, 0)),
                  pl.BlockSpec((32, 16), lambda i: (0, 0))],
        out_specs=[pl.BlockSpec((BN, D), lambda i: (i, 0)),
                   pl.BlockSpec((BN, 16), lambda i: (i, 0))],
        out_shape=[_S((NP, D), jnp.float32), _S((NP, 16), jnp.float32)],
    )(oa, ob, da, db, b1r, W2, A2p)


def _tc3_body(pa_ref, pb_ref, da_ref, db_ref, b_ref, w_ref, bh_ref, y_ref):
    den = da_ref[...] + db_ref[...]
    v = (pa_ref[...] + pb_ref[...]) / den[:, 0:1] + b_ref[...]
    x2 = jnp.where(v > 0, v, jnp.exp(v) - 1.0)
    y_ref[...] = jnp.dot(x2, w_ref[...],
                         preferred_element_type=jnp.float32) + bh_ref[...]


def _tc3(pa, pb, da, db, b2r, Whp, bhp):
    BN = 1264
    return pl.pallas_call(
        _tc3_body,
        grid=(NP // BN,),
        in_specs=[pl.BlockSpec((BN, 32), lambda i: (i, 0)),
                  pl.BlockSpec((BN, 32), lambda i: (i, 0)),
                  pl.BlockSpec((BN, 8), lambda i: (i, 0)),
                  pl.BlockSpec((BN, 8), lambda i: (i, 0)),
                  pl.BlockSpec((1, 32), lambda i: (0, 0)),
                  pl.BlockSpec((32, 8), lambda i: (0, 0)),
                  pl.BlockSpec((1, 8), lambda i: (0, 0))],
        out_specs=pl.BlockSpec((BN, 8), lambda i: (i, 0)),
        out_shape=_S((NP, 8), jnp.float32),
    )(pa, pb, da, db, b2r, Whp, bhp)


@_lazy_kernel(
    out_type=[_S((EP, 8), jnp.float32),
              _S((NP, 8), jnp.float32),
              _S((NP, 8), jnp.float32)],
    scratch_types=[pltpu.VMEM((B,), jnp.int32),
                   pltpu.VMEM((B,), jnp.int32),
                   pltpu.VMEM((B, 8), jnp.float32),
                   pltpu.VMEM((B, 8), jnp.float32),
                   pltpu.VMEM_SHARED((NP, 8), jnp.float32),
                   pltpu.VMEM_SHARED((NP, 8), jnp.float32),
                   pltpu.VMEM_SHARED((NP, 8), jnp.float32)],
)
def _sca(src_ref, dst_ref, as_ref, ad_ref,
         s_ref, dena_ref, denb_ref,
         idx_s, idx_d, asr, adr, as_sh, ad_sh, den_sh):
    c = lax.axis_index("c")
    s = lax.axis_index("s")
    w = s * NC + c
    ln = lax.iota(jnp.int32, 16)
    row_off = ln >> 3
    col = ln & 7
    zero16 = jnp.zeros((16,), jnp.float32)

    for (o, n) in _tile_slices(s):
        r = pl.ds(s * RPT + o, n)
        pltpu.sync_copy(as_ref.at[r], asr.at[pl.ds(0, n)])
        pltpu.sync_copy(asr.at[pl.ds(0, n)], as_sh.at[r])
        pltpu.sync_copy(ad_ref.at[r], adr.at[pl.ds(0, n)])
        pltpu.sync_copy(adr.at[pl.ds(0, n)], ad_sh.at[r])

    def zrow(j, _):
        plsc.store_scatter(asr, [row_off + 2 * j, col], zero16)
        return 0
    lax.fori_loop(0, 64, zrow, 0, unroll=8)
    for (o, n) in _tile_slices(s):
        pltpu.sync_copy(asr.at[pl.ds(0, n)],
                        den_sh.at[pl.ds(s * RPT + o, n)])
    plsc.subcore_barrier()

    def chunk(k, _):
        base = (w * CH + k) * B
        pltpu.sync_copy(src_ref.at[pl.ds(base, B)], idx_s)
        pltpu.sync_copy(dst_ref.at[pl.ds(base, B)], idx_d)
        pltpu.sync_copy(as_sh.at[idx_s], asr)
        pltpu.sync_copy(ad_sh.at[idx_d], adr)

        def grp(j, _):
            r = row_off + 2 * j
            a = (plsc.load_gather(asr, [r, col])
                 + plsc.load_gather(adr, [r, col]))
            a = jnp.exp(jnp.where(a >= 0, a, 0.2 * a))
            plsc.store_scatter(asr, [r, col], a)
            return 0
        lax.fori_loop(0, 64, grp, 0, unroll=8)
        pltpu.sync_copy(asr, s_ref.at[pl.ds(base, B)])
        pltpu.sync_copy(asr, den_sh.at[idx_d], add=True)
        return 0
    lax.fori_loop(0, CH, chunk, 0)
    plsc.subcore_barrier()

    for (o, n) in _tile_slices(s):
        r = pl.ds(s * RPT + o, n)
        pltpu.sync_copy(den_sh.at[r], asr.at[pl.ds(0, n)])

        @pl.when(c == 0)
        def _():
            pltpu.sync_copy(asr.at[pl.ds(0, n)], dena_ref.at[r])

        @pl.when(c == 1)
        def _():
            pltpu.sync_copy(asr.at[pl.ds(0, n)], denb_ref.at[r])


def _make_scb(nh, cw):
    @_lazy_kernel(
        out_type=[_S((NP, D), jnp.float32), _S((NP, D), jnp.float32)],
        scratch_types=[pltpu.VMEM((B,), jnp.int32),
                       pltpu.VMEM((B,), jnp.int32),
                       pltpu.VMEM((B, D), jnp.float32),
                       pltpu.VMEM((B, 8), jnp.float32),
                       pltpu.VMEM_SHARED((NP, D), jnp.float32)],
    )
    def _scb(src_ref, dst_ref, coef_ref, h_ref,
             oa_ref, ob_ref,
             idx_s, idx_d, hrows, crow, out_sh):
        c = lax.axis_index("c")
        s = lax.axis_index("s")
        w = s * NC + c
        zero16 = jnp.zeros((16,), jnp.float32)

        def zrow(i, _):
            for j in range(8):
                hrows[i, pl.ds(16 * j, 16)] = zero16
            return 0
        lax.fori_loop(0, B, zrow, 0, unroll=4)
        for (o, n) in _tile_slices(s):
            pltpu.sync_copy(hrows.at[pl.ds(0, n)],
                            out_sh.at[pl.ds(s * RPT + o, n)])
        plsc.subcore_barrier()

        def chunk(k, _):
            base = (w * CH + k) * B
            pltpu.sync_copy(src_ref.at[pl.ds(base, B)], idx_s)
            pltpu.sync_copy(dst_ref.at[pl.ds(base, B)], idx_d)
            pltpu.sync_copy(h_ref.at[idx_s], hrows)
            pltpu.sync_copy(coef_ref.at[pl.ds(base, B)], crow)

            def wrow(b, _):
                bf = jnp.full((16,), b, jnp.int32)
                for h in range(nh):
                    c16 = plsc.load_gather(
                        crow, [bf, jnp.full((16,), h, jnp.int32)])
                    for q in range(cw // 16):
                        o = h * cw + 16 * q
                        hrows[b, pl.ds(o, 16)] = hrows[b, pl.ds(o, 16)] * c16
                return 0
            lax.fori_loop(0, B, wrow, 0, unroll=2)
            pltpu.sync_copy(hrows, out_sh.at[idx_d], add=True)
            return 0
        lax.fori_loop(0, CH, chunk, 0)
        plsc.subcore_barrier()

        for (o, n) in _tile_slices(s):
            r = pl.ds(s * RPT + o, n)
            pltpu.sync_copy(out_sh.at[r], hrows.at[pl.ds(0, n)])

            @pl.when(c == 0)
            def _():
                pltpu.sync_copy(hrows.at[pl.ds(0, n)], oa_ref.at[r])

            @pl.when(c == 1)
            def _():
                pltpu.sync_copy(hrows.at[pl.ds(0, n)], ob_ref.at[r])
    return _scb


_scb1 = _make_scb(8, 16)
_scb2 = _make_scb(1, 32)


def _block_diag(att, heads, ch):
    a = att.reshape(heads, ch)
    return (a[:, :, None] * jnp.eye(heads, dtype=a.dtype)[:, None, :]
            ).reshape(heads * ch, heads)


def kernel(x, edge_index, W1, att_src1, att_dst1, b1,
           W2, att_src2, att_dst2, b2, Wh, bh):
    loop = jnp.arange(N, dtype=edge_index.dtype)
    src = jnp.concatenate([edge_index[0], loop,
                           jnp.zeros((EP - E - N,), jnp.int32)])
    dst = jnp.concatenate([edge_index[1], loop,
                           jnp.full((EP - E - N,), N, jnp.int32)])
    xp = jnp.pad(x, ((0, NP - N), (0, 0)))

    A1 = jnp.concatenate([_block_diag(att_src1, 8, 16),
                          _block_diag(att_dst1, 8, 16)], axis=1)
    h1, aa1 = _tc1(xp, W1, A1)
    as1 = aa1[:, :8]
    ad1 = aa1[:, 8:]

    s1, d1a, d1b = _sca(src, dst, as1, ad1)
    oa, ob = _scb1(src, dst, s1, h1)

    A2p = jnp.zeros((32, 16), jnp.float32)
    A2p = A2p.at[:, 0].set(att_src2.reshape(32)).at[:, 8].set(
        att_dst2.reshape(32))
    h2p, aa2 = _tc2(oa, ob, d1a, d1b, b1.reshape(1, D), W2, A2p)
    as2 = aa2[:, :8]
    ad2 = aa2[:, 8:]

    s2, d2a, d2b = _sca(src, dst, as2, ad2)
    o2a, o2b = _scb2(src, dst, s2, h2p)

    Whp = jnp.pad(Wh, ((0, 0), (0, 7)))
    bhp = jnp.broadcast_to(bh.reshape(1, 1), (1, 8))
    y8 = _tc3(o2a[:, :32], o2b[:, :32], d2a, d2b, b2.reshape(1, 32),
              Whp, bhp)
    return y8[:N, 0:1]
